# Initial kernel scaffold; baseline (speedup 1.0000x reference)
import jax
import jax.numpy as jnp
from jax.experimental import pallas as pl


def kernel(Z, W_m2v, W_mag, W_oli, P_m2v_w, P_m2v_b, P_mag_w, P_mag_b, P_oli_w, P_oli_b):
    raise NotImplementedError("write your pallas kernel here")


# speedup vs baseline: 3.8244x; 3.8244x over previous
import functools

import jax
import jax.numpy as jnp
from jax import lax
from jax.experimental import pallas as pl
from jax.experimental.pallas import tpu as pltpu
from jax.experimental.pallas import tpu_sc as plsc

D_MODEL = 64
CHUNK = 512


def _project_body(w1, w2, w3, p1, p2, p3, b1, b2, b3, t1, t2, t3):
    dn = (((1,), (1,)), ((), ()))
    t1[...] = lax.dot_general(w1[...], p1[...], dn,
                              preferred_element_type=jnp.float32) + b1[...]
    t2[...] = lax.dot_general(w2[...], p2[...], dn,
                              preferred_element_type=jnp.float32) + b2[...]
    t3[...] = lax.dot_general(w3[...], p3[...], dn,
                              preferred_element_type=jnp.float32) + b3[...]


def _project_tables(W_m2v, W_mag, W_oli, P1, P2, P3, b1, b2, b3):
    vocab = W_m2v.shape[0]
    out = jax.ShapeDtypeStruct((vocab, D_MODEL), jnp.float32)
    return pl.pallas_call(
        _project_body,
        out_shape=(out, out, out),
    )(W_m2v, W_mag, W_oli, P1, P2, P3,
      b1.reshape(1, D_MODEL), b2.reshape(1, D_MODEL), b3.reshape(1, D_MODEL))


def _gather_body(nc, nw, per_w, t1, t2, t3, zidx, o1, o2, o3,
                 idx_v, buf1, buf2, buf3, sem):
    wid = lax.axis_index("s") * nc + lax.axis_index("c")

    def step(i, carry):
        base = wid * per_w + i * CHUNK
        pltpu.sync_copy(zidx.at[pl.ds(base, CHUNK)], idx_v)
        c1 = pltpu.async_copy(t1.at[idx_v], buf1, sem)
        c2 = pltpu.async_copy(t2.at[idx_v], buf2, sem)
        c3 = pltpu.async_copy(t3.at[idx_v], buf3, sem)
        c1.wait()
        c2.wait()
        c3.wait()
        pltpu.sync_copy(buf1, o1.at[pl.ds(base, CHUNK)])
        pltpu.sync_copy(buf2, o2.at[pl.ds(base, CHUNK)])
        pltpu.sync_copy(buf3, o3.at[pl.ds(base, CHUNK)])
        return carry

    lax.fori_loop(0, per_w // CHUNK, step, 0)


def _gather_rows(T1, T2, T3, zflat):
    info = plsc.get_sparse_core_info()
    nc, ns = info.num_cores, info.num_subcores
    nw = nc * ns
    n = zflat.shape[0]
    per_w = n // nw
    out = jax.ShapeDtypeStruct((n, D_MODEL), jnp.float32)
    mesh = plsc.VectorSubcoreMesh(core_axis_name="c", subcore_axis_name="s")
    kfn = functools.partial(
        pl.kernel,
        mesh=mesh,
        compiler_params=pltpu.CompilerParams(use_tc_tiling_on_sc=False),
        out_type=(out, out, out),
        scratch_types=[
            pltpu.VMEM((CHUNK,), jnp.int32),
            pltpu.VMEM((CHUNK, D_MODEL), jnp.float32),
            pltpu.VMEM((CHUNK, D_MODEL), jnp.float32),
            pltpu.VMEM((CHUNK, D_MODEL), jnp.float32),
            pltpu.SemaphoreType.DMA,
        ],
    )(functools.partial(_gather_body, nc, nw, per_w))
    return kfn(T1, T2, T3, zflat)


def kernel(Z, W_m2v, W_mag, W_oli, P_m2v_w, P_m2v_b, P_mag_w, P_mag_b,
           P_oli_w, P_oli_b):
    B, S = Z.shape
    T1, T2, T3 = _project_tables(W_m2v, W_mag, W_oli,
                                 P_m2v_w, P_mag_w, P_oli_w,
                                 P_m2v_b, P_mag_b, P_oli_b)
    zflat = Z.reshape(-1).astype(jnp.int32)
    o1, o2, o3 = _gather_rows(T1, T2, T3, zflat)
    return (o1.reshape(B, S, D_MODEL),
            o2.reshape(B, S, D_MODEL),
            o3.reshape(B, S, D_MODEL))


# speedup vs baseline: 3.8444x; 1.0052x over previous
import functools

import jax
import jax.numpy as jnp
from jax import lax
from jax.experimental import pallas as pl
from jax.experimental.pallas import tpu as pltpu
from jax.experimental.pallas import tpu_sc as plsc

D_MODEL = 64
CHUNK = 256


def _project_body(w1, w2, w3, p1, p2, p3, b1, b2, b3, t1, t2, t3):
    dn = (((1,), (1,)), ((), ()))
    t1[...] = lax.dot_general(w1[...], p1[...], dn,
                              preferred_element_type=jnp.float32) + b1[...]
    t2[...] = lax.dot_general(w2[...], p2[...], dn,
                              preferred_element_type=jnp.float32) + b2[...]
    t3[...] = lax.dot_general(w3[...], p3[...], dn,
                              preferred_element_type=jnp.float32) + b3[...]


def _project_tables(W_m2v, W_mag, W_oli, P1, P2, P3, b1, b2, b3):
    vocab = W_m2v.shape[0]
    out = jax.ShapeDtypeStruct((vocab, D_MODEL), jnp.float32)
    return pl.pallas_call(
        _project_body,
        out_shape=(out, out, out),
    )(W_m2v, W_mag, W_oli, P1, P2, P3,
      b1.reshape(1, D_MODEL), b2.reshape(1, D_MODEL), b3.reshape(1, D_MODEL))


def _gather_body(nc, nw, per_w, t1, t2, t3, zidx, o1, o2, o3,
                 idx_v, b00, b01, b02, b10, b11, b12, gs0, gs1, ws0, ws1):
    wid = lax.axis_index("s") * nc + lax.axis_index("c")
    base = wid * per_w
    nchunks = per_w // CHUNK
    bufs = ((b00, b01, b02), (b10, b11, b12))
    gsems = (gs0, gs1)
    wsems = (ws0, ws1)
    outs = (o1, o2, o3)
    tabs = (t1, t2, t3)

    pltpu.sync_copy(zidx.at[pl.ds(base, per_w)], idx_v)

    def fire_gather(par, i):
        idx = idx_v.at[pl.ds(i * CHUNK, CHUNK)]
        for t in range(3):
            pltpu.async_copy(tabs[t].at[idx], bufs[par][t], gsems[par])

    def wait_gather(par):
        for t in range(3):
            pltpu.make_async_copy(tabs[t].at[idx_v.at[pl.ds(0, CHUNK)]],
                                  bufs[par][t], gsems[par]).wait()

    def fire_write(par, i):
        for t in range(3):
            pltpu.async_copy(bufs[par][t],
                             outs[t].at[pl.ds(base + i * CHUNK, CHUNK)],
                             wsems[par])

    def wait_write(par):
        for t in range(3):
            pltpu.make_async_copy(bufs[par][t],
                                  outs[t].at[pl.ds(base, CHUNK)],
                                  wsems[par]).wait()

    fire_gather(0, 0)

    def step(half, carry):
        i = half * 2
        wait_gather(0)
        fire_write(0, i)

        @pl.when(i > 0)
        def _():
            wait_write(1)

        fire_gather(1, i + 1)
        wait_gather(1)
        fire_write(1, i + 1)
        wait_write(0)

        @pl.when(i + 2 < nchunks)
        def _():
            fire_gather(0, i + 2)

        return carry

    lax.fori_loop(0, nchunks // 2, step, 0, unroll=False)
    wait_write(1)


def _gather_rows(T1, T2, T3, zflat):
    info = plsc.get_sparse_core_info()
    nc, ns = info.num_cores, info.num_subcores
    nw = nc * ns
    n = zflat.shape[0]
    per_w = n // nw
    out = jax.ShapeDtypeStruct((n, D_MODEL), jnp.float32)
    buf = pltpu.VMEM((CHUNK, D_MODEL), jnp.float32)
    mesh = plsc.VectorSubcoreMesh(core_axis_name="c", subcore_axis_name="s")
    kfn = functools.partial(
        pl.kernel,
        mesh=mesh,
        compiler_params=pltpu.CompilerParams(use_tc_tiling_on_sc=False),
        out_type=(out, out, out),
        scratch_types=[
            pltpu.VMEM((per_w,), jnp.int32),
            buf, buf, buf, buf, buf, buf,
            pltpu.SemaphoreType.DMA,
            pltpu.SemaphoreType.DMA,
            pltpu.SemaphoreType.DMA,
            pltpu.SemaphoreType.DMA,
        ],
    )(functools.partial(_gather_body, nc, nw, per_w))
    return kfn(T1, T2, T3, zflat)


def kernel(Z, W_m2v, W_mag, W_oli, P_m2v_w, P_m2v_b, P_mag_w, P_mag_b,
           P_oli_w, P_oli_b):
    B, S = Z.shape
    T1, T2, T3 = _project_tables(W_m2v, W_mag, W_oli,
                                 P_m2v_w, P_mag_w, P_oli_w,
                                 P_m2v_b, P_mag_b, P_oli_b)
    zflat = Z.reshape(-1).astype(jnp.int32)
    o1, o2, o3 = _gather_rows(T1, T2, T3, zflat)
    return (o1.reshape(B, S, D_MODEL),
            o2.reshape(B, S, D_MODEL),
            o3.reshape(B, S, D_MODEL))


# speedup vs baseline: 4.0983x; 1.0660x over previous
import functools

import jax
import jax.numpy as jnp
from jax import lax
from jax.experimental import pallas as pl
from jax.experimental.pallas import tpu as pltpu
from jax.experimental.pallas import tpu_sc as plsc

D_MODEL = 64
CHUNK = 256


def _project_body(w1, w2, w3, p1, p2, p3, b1, b2, b3, tab):
    dn = (((1,), (1,)), ((), ()))
    t1 = lax.dot_general(w1[...], p1[...], dn,
                         preferred_element_type=jnp.float32) + b1[...]
    t2 = lax.dot_general(w2[...], p2[...], dn,
                         preferred_element_type=jnp.float32) + b2[...]
    t3 = lax.dot_general(w3[...], p3[...], dn,
                         preferred_element_type=jnp.float32) + b3[...]
    tab[...] = jnp.concatenate([t1, t2, t3], axis=1)


def _project_tables(W_m2v, W_mag, W_oli, P1, P2, P3, b1, b2, b3):
    vocab = W_m2v.shape[0]
    return pl.pallas_call(
        _project_body,
        out_shape=jax.ShapeDtypeStruct((vocab, 3 * D_MODEL), jnp.float32),
    )(W_m2v, W_mag, W_oli, P1, P2, P3,
      b1.reshape(1, D_MODEL), b2.reshape(1, D_MODEL), b3.reshape(1, D_MODEL))


def _gather_body(nc, nw, per_w, tab, zidx, o1, o2, o3,
                 idx_v, buf0, buf1, gs0, gs1, ws0, ws1):
    wid = lax.axis_index("s") * nc + lax.axis_index("c")
    base = wid * per_w
    nchunks = per_w // CHUNK
    bufs = (buf0, buf1)
    gsems = (gs0, gs1)
    wsems = (ws0, ws1)
    outs = (o1, o2, o3)

    pltpu.sync_copy(zidx.at[pl.ds(base, per_w)], idx_v)

    def fire_gather(par, i):
        idx = idx_v.at[pl.ds(i * CHUNK, CHUNK)]
        pltpu.async_copy(tab.at[idx], bufs[par], gsems[par])

    def wait_gather(par):
        pltpu.make_async_copy(tab.at[idx_v.at[pl.ds(0, CHUNK)]],
                              bufs[par], gsems[par]).wait()

    def fire_write(par, i):
        for t in range(3):
            pltpu.async_copy(bufs[par].at[:, pl.ds(t * D_MODEL, D_MODEL)],
                             outs[t].at[pl.ds(base + i * CHUNK, CHUNK)],
                             wsems[par])

    def wait_write(par):
        for t in range(3):
            pltpu.make_async_copy(bufs[par].at[:, pl.ds(t * D_MODEL, D_MODEL)],
                                  outs[t].at[pl.ds(base, CHUNK)],
                                  wsems[par]).wait()

    fire_gather(0, 0)

    def step(half, carry):
        i = half * 2
        wait_gather(0)
        fire_write(0, i)

        @pl.when(i > 0)
        def _():
            wait_write(1)

        fire_gather(1, i + 1)
        wait_gather(1)
        fire_write(1, i + 1)
        wait_write(0)

        @pl.when(i + 2 < nchunks)
        def _():
            fire_gather(0, i + 2)

        return carry

    lax.fori_loop(0, nchunks // 2, step, 0, unroll=False)
    wait_write(1)


def _gather_rows(TAB, zflat):
    info = plsc.get_sparse_core_info()
    nc, ns = info.num_cores, info.num_subcores
    nw = nc * ns
    n = zflat.shape[0]
    per_w = n // nw
    out = jax.ShapeDtypeStruct((n, D_MODEL), jnp.float32)
    buf = pltpu.VMEM((CHUNK, 3 * D_MODEL), jnp.float32)
    mesh = plsc.VectorSubcoreMesh(core_axis_name="c", subcore_axis_name="s")
    kfn = functools.partial(
        pl.kernel,
        mesh=mesh,
        compiler_params=pltpu.CompilerParams(use_tc_tiling_on_sc=False),
        out_type=(out, out, out),
        scratch_types=[
            pltpu.VMEM((per_w,), jnp.int32),
            buf, buf,
            pltpu.SemaphoreType.DMA,
            pltpu.SemaphoreType.DMA,
            pltpu.SemaphoreType.DMA,
            pltpu.SemaphoreType.DMA,
        ],
    )(functools.partial(_gather_body, nc, nw, per_w))
    return kfn(TAB, zflat)


def kernel(Z, W_m2v, W_mag, W_oli, P_m2v_w, P_m2v_b, P_mag_w, P_mag_b,
           P_oli_w, P_oli_b):
    B, S = Z.shape
    TAB = _project_tables(W_m2v, W_mag, W_oli,
                          P_m2v_w, P_mag_w, P_oli_w,
                          P_m2v_b, P_mag_b, P_oli_b)
    zflat = Z.reshape(-1).astype(jnp.int32)
    o1, o2, o3 = _gather_rows(TAB, zflat)
    return (o1.reshape(B, S, D_MODEL),
            o2.reshape(B, S, D_MODEL),
            o3.reshape(B, S, D_MODEL))


# speedup vs baseline: 5.4223x; 1.3231x over previous
import functools

import jax
import jax.numpy as jnp
from jax import lax
from jax.experimental import pallas as pl
from jax.experimental.pallas import tpu as pltpu
from jax.experimental.pallas import tpu_sc as plsc

D_MODEL = 64
CHUNK = 256


def _project_body(w1, w2, w3, p1, p2, p3, b1, b2, b3, tab):
    dn = (((1,), (1,)), ((), ()))
    t1 = lax.dot_general(w1[...], p1[...], dn,
                         preferred_element_type=jnp.float32) + b1[...]
    t2 = lax.dot_general(w2[...], p2[...], dn,
                         preferred_element_type=jnp.float32) + b2[...]
    t3 = lax.dot_general(w3[...], p3[...], dn,
                         preferred_element_type=jnp.float32) + b3[...]
    tab[...] = jnp.concatenate([t1, t2, t3], axis=1)


def _project_tables(W_m2v, W_mag, W_oli, P1, P2, P3, b1, b2, b3):
    vocab = W_m2v.shape[0]
    return pl.pallas_call(
        _project_body,
        out_shape=jax.ShapeDtypeStruct((vocab, 3 * D_MODEL), jnp.float32),
    )(W_m2v, W_mag, W_oli, P1, P2, P3,
      b1.reshape(1, D_MODEL), b2.reshape(1, D_MODEL), b3.reshape(1, D_MODEL))


def _gather_body(nc, nw, per_w, tab, zidx, o1, o2, o3,
                 idx_v, tab_s, buf0, buf1, gs0, gs1, ws0, ws1):
    wid = lax.axis_index("s") * nc + lax.axis_index("c")
    base = wid * per_w
    nchunks = per_w // CHUNK
    bufs = (buf0, buf1)
    gsems = (gs0, gs1)
    wsems = (ws0, ws1)
    outs = (o1, o2, o3)

    @pl.when(lax.axis_index("s") == 0)
    def _():
        pltpu.sync_copy(tab, tab_s)

    plsc.subcore_barrier()

    pltpu.sync_copy(zidx.at[pl.ds(base, per_w)], idx_v)

    def fire_gather(par, i):
        idx = idx_v.at[pl.ds(i * CHUNK, CHUNK)]
        pltpu.async_copy(tab_s.at[idx], bufs[par], gsems[par])

    def wait_gather(par):
        pltpu.make_async_copy(tab_s.at[idx_v.at[pl.ds(0, CHUNK)]],
                              bufs[par], gsems[par]).wait()

    def fire_write(par, i):
        for t in range(3):
            pltpu.async_copy(bufs[par].at[:, pl.ds(t * D_MODEL, D_MODEL)],
                             outs[t].at[pl.ds(base + i * CHUNK, CHUNK)],
                             wsems[par])

    def wait_write(par):
        for t in range(3):
            pltpu.make_async_copy(bufs[par].at[:, pl.ds(t * D_MODEL, D_MODEL)],
                                  outs[t].at[pl.ds(base, CHUNK)],
                                  wsems[par]).wait()

    fire_gather(0, 0)

    def step(half, carry):
        i = half * 2
        wait_gather(0)
        fire_write(0, i)

        @pl.when(i > 0)
        def _():
            wait_write(1)

        fire_gather(1, i + 1)
        wait_gather(1)
        fire_write(1, i + 1)
        wait_write(0)

        @pl.when(i + 2 < nchunks)
        def _():
            fire_gather(0, i + 2)

        return carry

    lax.fori_loop(0, nchunks // 2, step, 0, unroll=False)
    wait_write(1)


def _gather_rows(TAB, zflat):
    info = plsc.get_sparse_core_info()
    nc, ns = info.num_cores, info.num_subcores
    nw = nc * ns
    n = zflat.shape[0]
    per_w = n // nw
    out = jax.ShapeDtypeStruct((n, D_MODEL), jnp.float32)
    buf = pltpu.VMEM((CHUNK, 3 * D_MODEL), jnp.float32)
    mesh = plsc.VectorSubcoreMesh(core_axis_name="c", subcore_axis_name="s")
    kfn = functools.partial(
        pl.kernel,
        mesh=mesh,
        compiler_params=pltpu.CompilerParams(use_tc_tiling_on_sc=False),
        out_type=(out, out, out),
        scratch_types=[
            pltpu.VMEM((per_w,), jnp.int32),
            pltpu.VMEM_SHARED((119, 3 * D_MODEL), jnp.float32),
            buf, buf,
            pltpu.SemaphoreType.DMA,
            pltpu.SemaphoreType.DMA,
            pltpu.SemaphoreType.DMA,
            pltpu.SemaphoreType.DMA,
        ],
    )(functools.partial(_gather_body, nc, nw, per_w))
    return kfn(TAB, zflat)


def kernel(Z, W_m2v, W_mag, W_oli, P_m2v_w, P_m2v_b, P_mag_w, P_mag_b,
           P_oli_w, P_oli_b):
    B, S = Z.shape
    TAB = _project_tables(W_m2v, W_mag, W_oli,
                          P_m2v_w, P_mag_w, P_oli_w,
                          P_m2v_b, P_mag_b, P_oli_b)
    zflat = Z.reshape(-1).astype(jnp.int32)
    o1, o2, o3 = _gather_rows(TAB, zflat)
    return (o1.reshape(B, S, D_MODEL),
            o2.reshape(B, S, D_MODEL),
            o3.reshape(B, S, D_MODEL))
